# baseline (speedup 1.0000x reference)
import numpy as np
import jax
import jax.numpy as jnp
from jax import lax
from jax.experimental import pallas as pl
from jax.experimental.pallas import tpu as pltpu
from jax.experimental.pallas import tpu_sc as plsc

_N_FREQ = 5
_FREQS = np.array(
    [[1.0 / 1000.0 ** (2.0 * i / _N_FREQ)] * 2 for i in range(1, _N_FREQ + 1)],
    dtype=np.float32,
).reshape(1, -1)
_SHIFTS = np.array(
    [[0.0, np.pi / 2.0] for _ in range(1, _N_FREQ + 1)], dtype=np.float32
).reshape(1, -1)

_FQ2 = np.zeros((20, 2), dtype=np.float32)
_FQ2[0:10, 0] = _FREQS[0]
_FQ2[10:20, 1] = _FREQS[0]
_SH2 = np.concatenate([_SHIFTS[0], _SHIFTS[0]]).reshape(20, 1)

_INV2PI = float(np.float32(1.0 / (2.0 * np.pi)))
_PI2_HI = float(np.float32(2.0 * np.pi))
_PI2_LO = float(np.float32(2.0 * np.pi - np.float64(np.float32(2.0 * np.pi))))


def _sin_coeffs():
    r = np.linspace(-np.pi, np.pi, 20001)
    powers = (1, 3, 5, 7, 9, 11, 13)
    a = np.stack([r ** p for p in powers], axis=1)
    c, *_ = np.linalg.lstsq(a, np.sin(r), rcond=None)
    return [float(np.float32(v)) for v in c]


_SINC = _sin_coeffs()


def _fast_sin(x):
    k = jnp.round(x * _INV2PI)
    r = x - k * _PI2_HI
    r = r - k * _PI2_LO
    r2 = r * r
    p = jnp.float32(_SINC[6])
    for c in (_SINC[5], _SINC[4], _SINC[3], _SINC[2], _SINC[1], _SINC[0]):
        p = p * r2 + c
    return p * r

_BLK = 6400

_NTILES = 16
_LANE = 128
_ROWS = 160
_ROWS_LAST = 100
_FIRE = 8


def _mlp_body(ct_ref, motif_ref, fq2_ref, sh2_ref, wf_ref, bfw_ref,
              w1_ref, b1_ref, w2_ref, b2_ref, out_ref):
    ct = ct_ref[...]
    fq2 = fq2_ref[...]
    a = jnp.concatenate(
        [ct[0:1, :] * fq2[0:10, 0:1], ct[1:2, :] * fq2[10:20, 1:2]],
        axis=0) + sh2_ref[...]
    enc = _fast_sin(a)
    w = (jnp.sum(enc * wf_ref[...], axis=0, keepdims=True)
         + bfw_ref[0, 0])
    mt = lax.dot_general(w1_ref[...], motif_ref[...],
                         (((1,), (1,)), ((), ())),
                         precision=lax.Precision.HIGHEST,
                         preferred_element_type=jnp.float32)
    ht = jnp.maximum(mt * w + b1_ref[...], 0.0)
    out_ref[...] = (jnp.sum(ht * w2_ref[...], axis=0, keepdims=True)
                    + b2_ref[0, 0])


def _fragment_mlp(coordinates, motifcounts, W_fw, b_fw, W1, b1, W2, b2):
    n = coordinates.shape[0]
    nblk = n // _BLK
    e_row = pl.pallas_call(
        _mlp_body,
        grid=(nblk,),
        in_specs=[
            pl.BlockSpec((2, _BLK), lambda i: (0, i)),
            pl.BlockSpec((_BLK, 128), lambda i: (i, 0)),
            pl.BlockSpec((20, 2), lambda i: (0, 0)),
            pl.BlockSpec((20, 1), lambda i: (0, 0)),
            pl.BlockSpec((20, 1), lambda i: (0, 0)),
            pl.BlockSpec((1, 1), lambda i: (0, 0)),
            pl.BlockSpec((128, 128), lambda i: (0, 0)),
            pl.BlockSpec((128, 1), lambda i: (0, 0)),
            pl.BlockSpec((128, 1), lambda i: (0, 0)),
            pl.BlockSpec((1, 1), lambda i: (0, 0)),
        ],
        out_specs=pl.BlockSpec((1, _BLK), lambda i: (0, i)),
        out_shape=jax.ShapeDtypeStruct((1, n), jnp.float32),
    )(coordinates.T, motifcounts, jnp.asarray(_FQ2), jnp.asarray(_SH2),
      W_fw.reshape(20, 1), b_fw.reshape(1, 1), W1, b1.reshape(128, 1),
      W2.reshape(128, 1), b2.reshape(1, 1))
    return e_row.reshape(n, 1)


def _make_segment_sum(nsegs):
    slice_per_tile = nsegs // _NTILES
    mesh = plsc.VectorSubcoreMesh(core_axis_name="c", subcore_axis_name="s")

    def body(e_hbm, ix_hbm, bias_hbm, out_hbm,
             idx_v, vals_v, obuf, bias_v, acc, sem):
        core = lax.axis_index("c")
        sid = lax.axis_index("s")

        @pl.when(core == 0)
        def _():
            off = pl.multiple_of(sid * slice_per_tile, slice_per_tile)

            @pl.when(sid < _NTILES - 1)
            def _stage_full():
                row0 = pl.multiple_of(sid * _ROWS, _ROWS)
                pltpu.sync_copy(ix_hbm.at[pl.ds(row0, _ROWS)], idx_v)
                pltpu.sync_copy(e_hbm.at[pl.ds(row0, _ROWS)], vals_v)

            @pl.when(sid == _NTILES - 1)
            def _stage_last():
                row0 = (_NTILES - 1) * _ROWS
                pltpu.sync_copy(ix_hbm.at[pl.ds(row0, _ROWS_LAST)],
                                idx_v.at[pl.ds(0, _ROWS_LAST)])
                pltpu.sync_copy(e_hbm.at[pl.ds(row0, _ROWS_LAST)],
                                vals_v.at[pl.ds(0, _ROWS_LAST)])

            zvec = jnp.zeros((16,), jnp.float32)

            def zero(j, carry):
                obuf[pl.ds(pl.multiple_of(j * 16, 16), 16)] = zvec
                return carry
            lax.fori_loop(0, slice_per_tile // 16, zero, 0)
            pltpu.sync_copy(obuf, acc.at[pl.ds(off, slice_per_tile)])
            plsc.subcore_barrier()

            def chunk(i, carry):
                base = i * _FIRE
                cps = [
                    pltpu.async_copy(vals_v.at[base + u],
                                     acc.at[idx_v.at[base + u]], sem, add=True)
                    for u in range(_FIRE)
                ]
                for cp in cps:
                    cp.wait()
                return carry
            nchunks = jnp.where(sid == _NTILES - 1, _ROWS_LAST // _FIRE,
                                _ROWS // _FIRE)
            lax.fori_loop(0, nchunks, chunk, 0)

            @pl.when(sid == _NTILES - 1)
            def _tail():
                base = (_ROWS_LAST // _FIRE) * _FIRE
                cps = [
                    pltpu.async_copy(vals_v.at[base + u],
                                     acc.at[idx_v.at[base + u]], sem, add=True)
                    for u in range(_ROWS_LAST - (_ROWS_LAST // _FIRE) * _FIRE)
                ]
                for cp in cps:
                    cp.wait()
            plsc.subcore_barrier()

            pltpu.sync_copy(bias_hbm, bias_v)
            pltpu.sync_copy(acc.at[pl.ds(off, slice_per_tile)], obuf)
            bvecs = [bias_v[pl.ds(v * 16, 16)] for v in range(8)]

            def add_bias(r, carry):
                row = pl.multiple_of(r * 128, 128)
                for v in range(8):
                    o = pl.multiple_of(row + v * 16, 16)
                    obuf[pl.ds(o, 16)] = obuf[pl.ds(o, 16)] + bvecs[v]
                return carry
            lax.fori_loop(0, slice_per_tile // 128, add_bias, 0)
            pltpu.sync_copy(obuf, out_hbm.at[pl.ds(off, slice_per_tile)])

    return pl.kernel(
        body,
        out_type=jax.ShapeDtypeStruct((nsegs,), jnp.float32),
        mesh=mesh,
        scratch_types=[
            pltpu.VMEM((_ROWS, _LANE), jnp.int32),
            pltpu.VMEM((_ROWS, _LANE), jnp.float32),
            pltpu.VMEM((slice_per_tile,), jnp.float32),
            pltpu.VMEM((_LANE,), jnp.float32),
            pltpu.VMEM_SHARED((nsegs,), jnp.float32),
            pltpu.SemaphoreType.DMA,
        ],
    )


def kernel(coordinates, motifcounts, local_cellxgene_ix, genes_oi, n_cells,
           n_genes, W_fw, b_fw, W1, b1, W2, b2, bias1):
    n = coordinates.shape[0]
    n_genes_static = genes_oi.shape[0]
    n_cells_static = 2048
    nsegs = n_cells_static * n_genes_static

    e = _fragment_mlp(coordinates, motifcounts, W_fw, b_fw, W1, b1, W2, b2)

    e_rows = e.reshape(n // _LANE, _LANE)
    ix_rows = local_cellxgene_ix.reshape(n // _LANE, _LANE)
    drift = ((n_cells - n_cells_static) + (n_genes - n_genes_static))
    bias_pg = jnp.take(bias1, genes_oi) + drift * jnp.float32(0.0)

    pooled = _make_segment_sum(nsegs)(e_rows, ix_rows, bias_pg)
    return pooled.reshape(n_cells_static, n_genes_static)


# speedup vs baseline: 1.0362x; 1.0362x over previous
import numpy as np
import jax
import jax.numpy as jnp
from jax import lax
from jax.experimental import pallas as pl
from jax.experimental.pallas import tpu as pltpu
from jax.experimental.pallas import tpu_sc as plsc

_N_FREQ = 5
_FREQS = np.array(
    [[1.0 / 1000.0 ** (2.0 * i / _N_FREQ)] * 2 for i in range(1, _N_FREQ + 1)],
    dtype=np.float32,
).reshape(1, -1)
_SHIFTS = np.array(
    [[0.0, np.pi / 2.0] for _ in range(1, _N_FREQ + 1)], dtype=np.float32
).reshape(1, -1)

_FQ2 = np.zeros((20, 2), dtype=np.float32)
_FQ2[0:10, 0] = _FREQS[0]
_FQ2[10:20, 1] = _FREQS[0]
_SH2 = np.concatenate([_SHIFTS[0], _SHIFTS[0]]).reshape(20, 1)

_INV2PI = float(np.float32(1.0 / (2.0 * np.pi)))
_PI2_HI = float(np.float32(2.0 * np.pi))
_PI2_LO = float(np.float32(2.0 * np.pi - np.float64(np.float32(2.0 * np.pi))))


def _sin_coeffs():
    r = np.linspace(-np.pi, np.pi, 20001)
    powers = (1, 3, 5, 7, 9, 11, 13)
    a = np.stack([r ** p for p in powers], axis=1)
    c, *_ = np.linalg.lstsq(a, np.sin(r), rcond=None)
    return [float(np.float32(v)) for v in c]


_SINC = _sin_coeffs()


def _fast_sin(x):
    k = jnp.round(x * _INV2PI)
    r = x - k * _PI2_HI
    r = r - k * _PI2_LO
    r2 = r * r
    p = jnp.float32(_SINC[6])
    for c in (_SINC[5], _SINC[4], _SINC[3], _SINC[2], _SINC[1], _SINC[0]):
        p = p * r2 + c
    return p * r

_BLK = 16000

_NTILES = 16
_LANE = 128
_ROWS = 160
_ROWS_LAST = 100
_FIRE = 8


def _mlp_body(ct_ref, motif_ref, fq2_ref, sh2_ref, wf_ref, bfw_ref,
              w1_ref, b1_ref, w2_ref, b2_ref, out_ref):
    ct = ct_ref[...]
    fq2 = fq2_ref[...]
    a = jnp.concatenate(
        [ct[0:1, :] * fq2[0:10, 0:1], ct[1:2, :] * fq2[10:20, 1:2]],
        axis=0) + sh2_ref[...]
    enc = _fast_sin(a)
    w = (jnp.sum(enc * wf_ref[...], axis=0, keepdims=True)
         + bfw_ref[0, 0])
    mt = lax.dot_general(w1_ref[...], motif_ref[...],
                         (((1,), (1,)), ((), ())),
                         precision=lax.Precision.HIGHEST,
                         preferred_element_type=jnp.float32)
    ht = jnp.maximum(mt * w + b1_ref[...], 0.0)
    out_ref[...] = (jnp.sum(ht * w2_ref[...], axis=0, keepdims=True)
                    + b2_ref[0, 0])


def _fragment_mlp(coordinates, motifcounts, W_fw, b_fw, W1, b1, W2, b2):
    n = coordinates.shape[0]
    nblk = n // _BLK
    e_row = pl.pallas_call(
        _mlp_body,
        grid=(nblk,),
        in_specs=[
            pl.BlockSpec((2, _BLK), lambda i: (0, i)),
            pl.BlockSpec((_BLK, 128), lambda i: (i, 0)),
            pl.BlockSpec((20, 2), lambda i: (0, 0)),
            pl.BlockSpec((20, 1), lambda i: (0, 0)),
            pl.BlockSpec((20, 1), lambda i: (0, 0)),
            pl.BlockSpec((1, 1), lambda i: (0, 0)),
            pl.BlockSpec((128, 128), lambda i: (0, 0)),
            pl.BlockSpec((128, 1), lambda i: (0, 0)),
            pl.BlockSpec((128, 1), lambda i: (0, 0)),
            pl.BlockSpec((1, 1), lambda i: (0, 0)),
        ],
        out_specs=pl.BlockSpec((1, _BLK), lambda i: (0, i)),
        out_shape=jax.ShapeDtypeStruct((1, n), jnp.float32),
    )(coordinates.T, motifcounts, jnp.asarray(_FQ2), jnp.asarray(_SH2),
      W_fw.reshape(20, 1), b_fw.reshape(1, 1), W1, b1.reshape(128, 1),
      W2.reshape(128, 1), b2.reshape(1, 1))
    return e_row.reshape(n, 1)


def _make_segment_sum(nsegs):
    slice_per_tile = nsegs // _NTILES
    mesh = plsc.VectorSubcoreMesh(core_axis_name="c", subcore_axis_name="s")

    def body(e_hbm, ix_hbm, bias_hbm, out_hbm,
             idx_v, vals_v, obuf, bias_v, acc, sem):
        core = lax.axis_index("c")
        sid = lax.axis_index("s")

        @pl.when(core == 0)
        def _():
            off = pl.multiple_of(sid * slice_per_tile, slice_per_tile)

            @pl.when(sid < _NTILES - 1)
            def _stage_full():
                row0 = pl.multiple_of(sid * _ROWS, _ROWS)
                pltpu.sync_copy(ix_hbm.at[pl.ds(row0, _ROWS)], idx_v)
                pltpu.sync_copy(e_hbm.at[pl.ds(row0, _ROWS)], vals_v)

            @pl.when(sid == _NTILES - 1)
            def _stage_last():
                row0 = (_NTILES - 1) * _ROWS
                pltpu.sync_copy(ix_hbm.at[pl.ds(row0, _ROWS_LAST)],
                                idx_v.at[pl.ds(0, _ROWS_LAST)])
                pltpu.sync_copy(e_hbm.at[pl.ds(row0, _ROWS_LAST)],
                                vals_v.at[pl.ds(0, _ROWS_LAST)])

            zvec = jnp.zeros((16,), jnp.float32)

            def zero(j, carry):
                obuf[pl.ds(pl.multiple_of(j * 16, 16), 16)] = zvec
                return carry
            lax.fori_loop(0, slice_per_tile // 16, zero, 0)
            pltpu.sync_copy(obuf, acc.at[pl.ds(off, slice_per_tile)])
            plsc.subcore_barrier()

            def chunk(i, carry):
                base = i * _FIRE
                cps = [
                    pltpu.async_copy(vals_v.at[base + u],
                                     acc.at[idx_v.at[base + u]], sem, add=True)
                    for u in range(_FIRE)
                ]
                for cp in cps:
                    cp.wait()
                return carry
            nchunks = jnp.where(sid == _NTILES - 1, _ROWS_LAST // _FIRE,
                                _ROWS // _FIRE)
            lax.fori_loop(0, nchunks, chunk, 0)

            @pl.when(sid == _NTILES - 1)
            def _tail():
                base = (_ROWS_LAST // _FIRE) * _FIRE
                cps = [
                    pltpu.async_copy(vals_v.at[base + u],
                                     acc.at[idx_v.at[base + u]], sem, add=True)
                    for u in range(_ROWS_LAST - (_ROWS_LAST // _FIRE) * _FIRE)
                ]
                for cp in cps:
                    cp.wait()
            plsc.subcore_barrier()

            pltpu.sync_copy(bias_hbm, bias_v)
            pltpu.sync_copy(acc.at[pl.ds(off, slice_per_tile)], obuf)
            bvecs = [bias_v[pl.ds(v * 16, 16)] for v in range(8)]

            def add_bias(r, carry):
                row = pl.multiple_of(r * 128, 128)
                for v in range(8):
                    o = pl.multiple_of(row + v * 16, 16)
                    obuf[pl.ds(o, 16)] = obuf[pl.ds(o, 16)] + bvecs[v]
                return carry
            lax.fori_loop(0, slice_per_tile // 128, add_bias, 0)
            pltpu.sync_copy(obuf, out_hbm.at[pl.ds(off, slice_per_tile)])

    return pl.kernel(
        body,
        out_type=jax.ShapeDtypeStruct((nsegs,), jnp.float32),
        mesh=mesh,
        scratch_types=[
            pltpu.VMEM((_ROWS, _LANE), jnp.int32),
            pltpu.VMEM((_ROWS, _LANE), jnp.float32),
            pltpu.VMEM((slice_per_tile,), jnp.float32),
            pltpu.VMEM((_LANE,), jnp.float32),
            pltpu.VMEM_SHARED((nsegs,), jnp.float32),
            pltpu.SemaphoreType.DMA,
        ],
    )


def kernel(coordinates, motifcounts, local_cellxgene_ix, genes_oi, n_cells,
           n_genes, W_fw, b_fw, W1, b1, W2, b2, bias1):
    n = coordinates.shape[0]
    n_genes_static = genes_oi.shape[0]
    n_cells_static = 2048
    nsegs = n_cells_static * n_genes_static

    e = _fragment_mlp(coordinates, motifcounts, W_fw, b_fw, W1, b1, W2, b2)

    e_rows = e.reshape(n // _LANE, _LANE)
    ix_rows = local_cellxgene_ix.reshape(n // _LANE, _LANE)
    drift = ((n_cells - n_cells_static) + (n_genes - n_genes_static))
    bias_pg = jnp.take(bias1, genes_oi) + drift * jnp.float32(0.0)

    pooled = _make_segment_sum(nsegs)(e_rows, ix_rows, bias_pg)
    return pooled.reshape(n_cells_static, n_genes_static)
